# baseline (speedup 1.0000x reference)
import functools

import jax
import jax.numpy as jnp
from jax import lax
from jax.experimental import pallas as pl
from jax.experimental.pallas import tpu as pltpu
from jax.experimental.pallas import tpu_sc as plsc

_NC = 2
_NS = 16
_NW = _NC * _NS
_IDX_CHUNK = 128


def _sc_gather_into_wide(table, idx, out_dim, col_off):
    B = idx.shape[0]
    D = table.shape[1]
    b_per_w = B // _NW
    n_chunks = b_per_w // _IDX_CHUNK
    idx3 = idx.reshape(_NW, n_chunks, _IDX_CHUNK)

    mesh = plsc.VectorSubcoreMesh(
        core_axis_name="c", subcore_axis_name="s",
        num_cores=_NC, num_subcores=_NS,
    )

    @functools.partial(
        pl.kernel,
        mesh=mesh,
        out_type=jax.ShapeDtypeStruct((B, out_dim), jnp.float32),
        scratch_types=[
            pltpu.VMEM((n_chunks, _IDX_CHUNK), jnp.int32),
            pltpu.VMEM((b_per_w, D), jnp.float32),
            pltpu.SemaphoreType.DMA,
        ],
    )
    def k(table_hbm, idx_hbm, out_hbm, idx_v, rows_v, sem):
        wid = lax.axis_index("s") * _NC + lax.axis_index("c")
        base = wid * b_per_w
        pltpu.sync_copy(idx_hbm.at[wid], idx_v)
        copies = []
        for c in range(n_chunks):
            cp = pltpu.make_async_copy(
                table_hbm.at[idx_v.at[c]],
                rows_v.at[pl.ds(c * _IDX_CHUNK, _IDX_CHUNK)],
                sem,
            )
            cp.start()
            copies.append(cp)
        for cp in copies:
            cp.wait()
        pltpu.sync_copy(rows_v,
                        out_hbm.at[pl.ds(base, b_per_w), pl.ds(col_off, D)])

    return k(table, idx3)


_INV_2PI = 0.15915493667125702
_RED_C1 = 6.283203125
_RED_C2 = -1.7642974853515625e-05
_RED_C3 = -1.7484555314695172e-07
_COS_POLY = (0.9999994437335175, -0.49999558241466635, 0.04166103364082131,
             -0.0013862750367048366, 2.4253235371477696e-05,
             -2.219415543283559e-07)


def _fast_cos(x):
    n = jnp.round(x * _INV_2PI)
    r = x - n * _RED_C1
    r = r - n * _RED_C2
    r = r - n * _RED_C3
    y = r * r
    acc = jnp.float32(_COS_POLY[-1])
    for c in _COS_POLY[-2::-1]:
        acc = acc * y + jnp.float32(c)
    return acc


def _fuse_body(node_ref, ts_ref, prev_ref, w_ref, b_ref, scout_ref, out_ref):
    j = pl.program_id(1)

    @pl.when(j < 2)
    def _copy_node():
        out_ref[...] = node_ref[...]

    @pl.when(j == 2)
    def _time_encode():
        dt = ts_ref[...] - prev_ref[...]
        out_ref[...] = _fast_cos(dt * w_ref[...] + b_ref[...])


def _tc_fuse(node_msgs, ts, prev_ts, time_w, time_b, sc_out, block_rows):
    B, msg = node_msgs.shape
    tdim = time_w.shape[0]
    out_dim = sc_out.shape[1]
    cb = 128
    grid = (B // block_rows, 3)
    return pl.pallas_call(
        _fuse_body,
        grid=grid,
        in_specs=[
            pl.BlockSpec((block_rows, cb),
                         lambda i, j: (i, jnp.minimum(j, 1))),
            pl.BlockSpec((block_rows, 1), lambda i, j: (i, 0)),
            pl.BlockSpec((block_rows, 1), lambda i, j: (i, 0)),
            pl.BlockSpec((1, tdim), lambda i, j: (0, 0)),
            pl.BlockSpec((1, tdim), lambda i, j: (0, 0)),
            pl.BlockSpec(memory_space=pl.ANY),
        ],
        out_specs=pl.BlockSpec((block_rows, cb),
                               lambda i, j: (i, j + (j == 2))),
        out_shape=jax.ShapeDtypeStruct((B, out_dim), jnp.float32),
        input_output_aliases={5: 0},
    )(node_msgs, ts.reshape(B, 1), prev_ts.reshape(B, 1),
      time_w.reshape(1, tdim), time_b.reshape(1, tdim), sc_out)


def kernel(node_msgs, eids, ts, prev_ts, edge_table, time_w, time_b):
    eids_i32 = eids.astype(jnp.int32)
    msg = node_msgs.shape[1]
    out_dim = msg + edge_table.shape[1] + time_w.shape[0]
    sc_out = _sc_gather_into_wide(edge_table, eids_i32, out_dim, col_off=msg)
    full_msgs = _tc_fuse(node_msgs, ts, prev_ts, time_w, time_b, sc_out,
                         block_rows=2048)
    return (full_msgs, ts)
